# Initial kernel scaffold; baseline (speedup 1.0000x reference)
import jax
import jax.numpy as jnp
from jax.experimental import pallas as pl


def kernel(species, embedding):
    raise NotImplementedError("write your pallas kernel here")


# speedup vs baseline: 1.0298x; 1.0298x over previous
import functools

import jax
import jax.numpy as jnp
from jax import lax
from jax.experimental import pallas as pl
from jax.experimental.pallas import tpu as pltpu
from jax.experimental.pallas import tpu_sc as plsc

B = 100000
D = 128
NW = 32
CHUNK = 128
CPW = 25
B_PAD = NW * CPW * CHUNK
FULL_CHUNKS = B // CHUNK
REM = B - FULL_CHUNKS * CHUNK
NBUF = 4


def _make_kernel():
    mesh = plsc.VectorSubcoreMesh(core_axis_name="c", subcore_axis_name="s")

    @functools.partial(
        pl.kernel,
        mesh=mesh,
        out_type=jax.ShapeDtypeStruct((B, D), jnp.float32),
        scratch_types=[
            pltpu.VMEM((CPW * CHUNK,), jnp.int32),
            pltpu.VMEM((NBUF, CHUNK, D), jnp.float32),
        ] + [pltpu.SemaphoreType.DMA] * NBUF,
    )
    def emb_kernel(idx_hbm, table_hbm, out_hbm, idx_v, rows_v, *gsems):
        wid = lax.axis_index("s") * 2 + lax.axis_index("c")
        base_chunk = wid * CPW

        pltpu.sync_copy(idx_hbm.at[pl.ds(base_chunk * CHUNK, CPW * CHUNK)], idx_v)

        def gather(j):
            b = j % NBUF
            return pltpu.async_copy(
                table_hbm.at[idx_v.at[pl.ds(j * CHUNK, CHUNK)]],
                rows_v.at[b],
                gsems[b],
            )

        handles = [gather(j) for j in range(NBUF)]

        for j in range(CPW):
            b = j % NBUF
            handles[b].wait()
            c = base_chunk + j
            row0 = c * CHUNK

            @pl.when(c < FULL_CHUNKS)
            def _():
                pltpu.sync_copy(rows_v.at[b], out_hbm.at[pl.ds(row0, CHUNK)])

            @pl.when(c == FULL_CHUNKS)
            def _():
                pltpu.sync_copy(
                    rows_v.at[b, pl.ds(0, REM)],
                    out_hbm.at[pl.ds(FULL_CHUNKS * CHUNK, REM)],
                )

            if j + NBUF < CPW:
                handles[b] = gather(j + NBUF)

    return emb_kernel


_emb = _make_kernel()


@jax.jit
def kernel(species, embedding):
    idx = jnp.concatenate(
        [species.astype(jnp.int32), jnp.zeros((B_PAD - B,), jnp.int32)]
    )
    return _emb(idx, embedding)


# speedup vs baseline: 5.4294x; 5.2724x over previous
import functools

import jax
import jax.numpy as jnp
from jax import lax
from jax.experimental import pallas as pl
from jax.experimental.pallas import tpu as pltpu
from jax.experimental.pallas import tpu_sc as plsc

B = 100000
D = 128
NW = 32
CHUNK = 128
CPW = 25
B_PAD = NW * CPW * CHUNK
FULL_CHUNKS = B // CHUNK
REM = B - FULL_CHUNKS * CHUNK
NBUF = 4


def _make_kernel():
    mesh = plsc.VectorSubcoreMesh(core_axis_name="c", subcore_axis_name="s")

    @functools.partial(
        pl.kernel,
        mesh=mesh,
        out_type=jax.ShapeDtypeStruct((B, D), jnp.float32),
        scratch_types=[
            pltpu.VMEM((CPW * CHUNK,), jnp.int32),
            pltpu.VMEM((NBUF, CHUNK, D), jnp.float32),
            pltpu.VMEM_SHARED((100, D), jnp.float32),
        ] + [pltpu.SemaphoreType.DMA] * NBUF,
    )
    def emb_kernel(idx_hbm, table_hbm, out_hbm, idx_v, rows_v, table_s, *gsems):
        wid = lax.axis_index("s") * 2 + lax.axis_index("c")
        base_chunk = wid * CPW

        @pl.when(lax.axis_index("s") == 0)
        def _():
            pltpu.sync_copy(table_hbm, table_s)

        plsc.subcore_barrier()
        pltpu.sync_copy(idx_hbm.at[pl.ds(base_chunk * CHUNK, CPW * CHUNK)], idx_v)

        def gather(j):
            b = j % NBUF
            return pltpu.async_copy(
                table_s.at[idx_v.at[pl.ds(j * CHUNK, CHUNK)]],
                rows_v.at[b],
                gsems[b],
            )

        handles = [gather(j) for j in range(NBUF)]

        for j in range(CPW):
            b = j % NBUF
            handles[b].wait()
            c = base_chunk + j
            row0 = c * CHUNK

            @pl.when(c < FULL_CHUNKS)
            def _():
                pltpu.sync_copy(rows_v.at[b], out_hbm.at[pl.ds(row0, CHUNK)])

            @pl.when(c == FULL_CHUNKS)
            def _():
                pltpu.sync_copy(
                    rows_v.at[b, pl.ds(0, REM)],
                    out_hbm.at[pl.ds(FULL_CHUNKS * CHUNK, REM)],
                )

            if j + NBUF < CPW:
                handles[b] = gather(j + NBUF)

    return emb_kernel


_emb = _make_kernel()


@jax.jit
def kernel(species, embedding):
    idx = jnp.concatenate(
        [species.astype(jnp.int32), jnp.zeros((B_PAD - B,), jnp.int32)]
    )
    return _emb(idx, embedding)


# speedup vs baseline: 5.4406x; 1.0021x over previous
import functools

import jax
import jax.numpy as jnp
from jax import lax
from jax.experimental import pallas as pl
from jax.experimental.pallas import tpu as pltpu
from jax.experimental.pallas import tpu_sc as plsc

B = 100000
D = 128
NW = 32
CHUNK = 128
CPW = 25
B_PAD = NW * CPW * CHUNK
FULL_CHUNKS = B // CHUNK
REM = B - FULL_CHUNKS * CHUNK
NBUF = 6
LOOKAHEAD = 3


def _make_kernel():
    mesh = plsc.VectorSubcoreMesh(core_axis_name="c", subcore_axis_name="s")

    @functools.partial(
        pl.kernel,
        mesh=mesh,
        out_type=jax.ShapeDtypeStruct((B, D), jnp.float32),
        scratch_types=[
            pltpu.VMEM((CPW * CHUNK,), jnp.int32),
            pltpu.VMEM((NBUF, CHUNK, D), jnp.float32),
            pltpu.VMEM_SHARED((100, D), jnp.float32),
        ] + [pltpu.SemaphoreType.DMA] * (2 * NBUF),
    )
    def emb_kernel(idx_hbm, table_hbm, out_hbm, idx_v, rows_v, table_s, *sems):
        gsems = sems[:NBUF]
        ssems = sems[NBUF:]
        wid = lax.axis_index("s") * 2 + lax.axis_index("c")
        base_chunk = wid * CPW

        @pl.when(lax.axis_index("s") == 0)
        def _():
            pltpu.sync_copy(table_hbm, table_s)

        plsc.subcore_barrier()
        pltpu.sync_copy(idx_hbm.at[pl.ds(base_chunk * CHUNK, CPW * CHUNK)], idx_v)

        def gather(j):
            b = j % NBUF
            return pltpu.async_copy(
                table_s.at[idx_v.at[pl.ds(j * CHUNK, CHUNK)]],
                rows_v.at[b],
                gsems[b],
            )

        def store_copies(j):
            b = j % NBUF
            c = base_chunk + j
            full = pltpu.make_async_copy(
                rows_v.at[b], out_hbm.at[pl.ds(c * CHUNK, CHUNK)], ssems[b]
            )
            part = pltpu.make_async_copy(
                rows_v.at[b, pl.ds(0, REM)],
                out_hbm.at[pl.ds(FULL_CHUNKS * CHUNK, REM)],
                ssems[b],
            )
            return c, full, part

        def store_start(j):
            c, full, part = store_copies(j)
            pl.when(c < FULL_CHUNKS)(full.start)
            pl.when(c == FULL_CHUNKS)(part.start)

        def store_wait(j):
            c, full, part = store_copies(j)
            pl.when(c < FULL_CHUNKS)(full.wait)
            pl.when(c == FULL_CHUNKS)(part.wait)

        ghandles = [None] * NBUF
        for j in range(LOOKAHEAD):
            ghandles[j % NBUF] = gather(j)

        for j in range(CPW):
            b = j % NBUF
            ghandles[b].wait()
            store_start(j)

            nj = j + LOOKAHEAD
            if nj < CPW:
                if nj >= NBUF:
                    store_wait(nj - NBUF)
                ghandles[nj % NBUF] = gather(nj)

        for j in range(CPW - NBUF, CPW):
            store_wait(j)

    return emb_kernel


_emb = _make_kernel()


@jax.jit
def kernel(species, embedding):
    idx = jnp.concatenate(
        [species.astype(jnp.int32), jnp.zeros((B_PAD - B,), jnp.int32)]
    )
    return _emb(idx, embedding)


# speedup vs baseline: 5.7116x; 1.0498x over previous
import functools

import jax
import jax.numpy as jnp
from jax import lax
from jax.experimental import pallas as pl
from jax.experimental.pallas import tpu as pltpu
from jax.experimental.pallas import tpu_sc as plsc

B = 100000
D = 128
V = 100
NW = 32
CHUNK = 128
CPW = 25
RPW = CPW * CHUNK
FULL_CHUNKS = B // CHUNK
REM = B - FULL_CHUNKS * CHUNK
NBUF = 5
NG = CPW // NBUF
LAST_VALID = B - (NW - 1) * RPW


def _make_kernel():
    mesh = plsc.VectorSubcoreMesh(core_axis_name="c", subcore_axis_name="s")

    @functools.partial(
        pl.kernel,
        mesh=mesh,
        out_type=jax.ShapeDtypeStruct((B, D), jnp.float32),
        scratch_types=[
            pltpu.VMEM((RPW,), jnp.int32),
            pltpu.VMEM((NBUF, CHUNK, D), jnp.float32),
            pltpu.VMEM_SHARED((V, D), jnp.float32),
            pltpu.SemaphoreType.DMA,
        ] + [pltpu.SemaphoreType.DMA] * (2 * NBUF),
    )
    def emb_kernel(idx_hbm, table_hbm, out_hbm, idx_v, rows_v, table_s,
                   isem, *sems):
        gsems = sems[:NBUF]
        ssems = sems[NBUF:]
        wid = lax.axis_index("s") * 2 + lax.axis_index("c")
        base_chunk = wid * CPW
        last = wid == NW - 1

        idx_full = pltpu.make_async_copy(
            idx_hbm.at[pl.ds(wid * RPW, RPW)], idx_v, isem
        )
        idx_part = pltpu.make_async_copy(
            idx_hbm.at[pl.ds(wid * RPW, LAST_VALID)],
            idx_v.at[pl.ds(0, LAST_VALID)],
            isem,
        )
        pl.when(~last)(idx_full.start)
        pl.when(last)(idx_part.start)

        @pl.when(lax.axis_index("s") == 0)
        def _():
            pltpu.sync_copy(table_hbm, table_s)

        pl.when(~last)(idx_full.wait)
        pl.when(last)(idx_part.wait)
        plsc.subcore_barrier()

        def gather_copies(j, b):
            c = base_chunk + j
            full = pltpu.make_async_copy(
                table_s.at[idx_v.at[pl.ds(j * CHUNK, CHUNK)]],
                rows_v.at[b],
                gsems[b],
            )
            part = pltpu.make_async_copy(
                table_s.at[idx_v.at[pl.ds(j * CHUNK, REM)]],
                rows_v.at[b, pl.ds(0, REM)],
                gsems[b],
            )
            return c, full, part

        def gather_start(j, b):
            c, full, part = gather_copies(j, b)
            pl.when(c < FULL_CHUNKS)(full.start)
            pl.when(c == FULL_CHUNKS)(part.start)

        def gather_wait(j, b):
            c, full, part = gather_copies(j, b)
            pl.when(c < FULL_CHUNKS)(full.wait)
            pl.when(c == FULL_CHUNKS)(part.wait)

        def store_copies(j, b):
            c = base_chunk + j
            full = pltpu.make_async_copy(
                rows_v.at[b], out_hbm.at[pl.ds(c * CHUNK, CHUNK)], ssems[b]
            )
            part = pltpu.make_async_copy(
                rows_v.at[b, pl.ds(0, REM)],
                out_hbm.at[pl.ds(FULL_CHUNKS * CHUNK, REM)],
                ssems[b],
            )
            return c, full, part

        def store_start(j, b):
            c, full, part = store_copies(j, b)
            pl.when(c < FULL_CHUNKS)(full.start)
            pl.when(c == FULL_CHUNKS)(part.start)

        def store_wait(j, b):
            c, full, part = store_copies(j, b)
            pl.when(c < FULL_CHUNKS)(full.wait)
            pl.when(c == FULL_CHUNKS)(part.wait)

        def body(g, carry):
            for b in range(NBUF):
                @pl.when(g > 0)
                def _():
                    store_wait((g - 1) * NBUF + b, b)

                gather_start(g * NBUF + b, b)
            for b in range(NBUF):
                gather_wait(g * NBUF + b, b)
                store_start(g * NBUF + b, b)
            return carry

        lax.fori_loop(0, NG, body, 0)
        for b in range(NBUF):
            store_wait((NG - 1) * NBUF + b, b)

    return emb_kernel


_emb = _make_kernel()


@jax.jit
def kernel(species, embedding):
    return _emb(species.astype(jnp.int32), embedding)
